# initial kernel scaffold (unmeasured)
import functools

import jax
import jax.numpy as jnp
from jax import lax
from jax.experimental import pallas as pl
from jax.experimental.pallas import tpu as pltpu

N_DEV = 4
B, Sq, Skv_per, Hq, Dh = 2, 128, 128, 4, 64
D_model = 512
D_qk = Hq * Dh


def kernel(x, Wq, K_ext, V_ext, Wo):
    def body(x_ref, wq_ref, k_ref, v_ref, wo_ref, out_ref,
             kv_ref, send_sems, recv_sems):
        my = lax.axis_index("i")

        barrier = pltpu.get_barrier_semaphore()
        for off in (1, 2, 3):
            peer = lax.rem(my + off, N_DEV)
            pl.semaphore_signal(
                barrier, inc=1,
                device_id=(peer,), device_id_type=pl.DeviceIdType.MESH,
            )
        pl.semaphore_wait(barrier, N_DEV - 1)

        kv_ref[0, 0] = k_ref[...].astype(jnp.bfloat16)
        kv_ref[0, 1] = v_ref[...].astype(jnp.bfloat16)

        rdmas = []
        for off in (1, 2, 3):
            peer = lax.rem(my + off, N_DEV)
            rdma = pltpu.make_async_remote_copy(
                src_ref=kv_ref.at[0],
                dst_ref=kv_ref.at[off],
                send_sem=send_sems.at[off - 1],
                recv_sem=recv_sems.at[off - 1],
                device_id=(peer,),
                device_id_type=pl.DeviceIdType.MESH,
            )
            rdma.start()
            rdmas.append(rdma)

        wq = wq_ref[...].astype(jnp.bfloat16)
        wo = wo_ref[...].astype(jnp.bfloat16)
        q_all = [
            lax.dot(x_ref[b].astype(jnp.bfloat16), wq,
                    preferred_element_type=jnp.float32)
            for b in range(B)
        ]
        qi = lax.broadcasted_iota(jnp.int32, (Sq, Skv_per), 0)
        kj = lax.broadcasted_iota(jnp.int32, (Sq, Skv_per), 1)

        for r in rdmas:
            r.wait_recv()

        for b in range(B):
            ctx_heads = []
            for h in range(Hq):
                qh = q_all[b][:, h * Dh:(h + 1) * Dh].astype(jnp.bfloat16)
                scores = []
                for s in range(N_DEV):
                    origin = lax.rem(my - s + N_DEV, N_DEV)
                    kg = kj + origin * Skv_per
                    kh = kv_ref[s, 0, b][:, h, :]
                    sc = lax.dot_general(
                        qh, kh, (((1,), (1,)), ((), ())),
                        preferred_element_type=jnp.float32,
                    ) * 0.125
                    mask = (jnp.abs(qi - kg) <= 128) | (kg < 32) | (qi < 32)
                    scores.append(jnp.where(mask, sc, -1e9))
                row_max = functools.reduce(
                    jnp.maximum,
                    [sc.max(axis=-1, keepdims=True) for sc in scores],
                )
                ws = [jnp.exp(sc - row_max) for sc in scores]
                denom = functools.reduce(
                    jnp.add, [w.sum(axis=-1, keepdims=True) for w in ws]
                )
                ctx = functools.reduce(jnp.add, [
                    lax.dot(ws[s].astype(jnp.bfloat16),
                            kv_ref[s, 1, b][:, h, :],
                            preferred_element_type=jnp.float32)
                    for s in range(N_DEV)
                ])
                ctx_heads.append(ctx / denom)
            ctx_b = jnp.concatenate(ctx_heads, axis=1)
            out_ref[b] = lax.dot(ctx_b.astype(jnp.bfloat16), wo,
                                 preferred_element_type=jnp.float32)

        for r in rdmas:
            r.wait_send()

    return pl.pallas_call(
        body,
        out_shape=jax.ShapeDtypeStruct((B, Sq, D_model), jnp.float32),
        in_specs=[pl.BlockSpec(memory_space=pltpu.VMEM)] * 5,
        out_specs=pl.BlockSpec(memory_space=pltpu.VMEM),
        scratch_shapes=[
            pltpu.VMEM((N_DEV, 2, B, Skv_per, Hq, Dh), jnp.bfloat16),
            pltpu.SemaphoreType.DMA((3,)),
            pltpu.SemaphoreType.DMA((3,)),
        ],
        compiler_params=pltpu.CompilerParams(collective_id=0),
    )(x, Wq, K_ext, V_ext, Wo)


# baseline (device time: 35789 ns/iter reference)
import functools

import jax
import jax.numpy as jnp
from jax import lax
from jax.experimental import pallas as pl
from jax.experimental.pallas import tpu as pltpu

N_DEV = 4
B, Sq, Skv_per, Hq, Dh = 2, 128, 128, 4, 64
_CDT = jnp.float32
D_model = 512
D_qk = Hq * Dh


def kernel(x, Wq, K_ext, V_ext, Wo):
    def body(x_ref, wq_ref, k_ref, v_ref, wo_ref, out_ref,
             kv_ref, send_sems, recv_sems):
        my = lax.axis_index("i")

        barrier = pltpu.get_barrier_semaphore()
        for off in (1, 2, 3):
            peer = lax.rem(my + off, N_DEV)
            pl.semaphore_signal(
                barrier, inc=1,
                device_id=(peer,), device_id_type=pl.DeviceIdType.MESH,
            )
        pl.semaphore_wait(barrier, N_DEV - 1)

        kv_ref[0, 0] = k_ref[...].astype(_CDT)
        kv_ref[0, 1] = v_ref[...].astype(_CDT)

        rdmas = []
        for off in (1, 2, 3):
            peer = lax.rem(my + off, N_DEV)
            rdma = pltpu.make_async_remote_copy(
                src_ref=kv_ref.at[0],
                dst_ref=kv_ref.at[off],
                send_sem=send_sems.at[off - 1],
                recv_sem=recv_sems.at[off - 1],
                device_id=(peer,),
                device_id_type=pl.DeviceIdType.MESH,
            )
            rdma.start()
            rdmas.append(rdma)

        wq = wq_ref[...].astype(_CDT)
        wo = wo_ref[...].astype(_CDT)
        q_all = [
            lax.dot(x_ref[b].astype(_CDT), wq,
                    preferred_element_type=jnp.float32)
            for b in range(B)
        ]
        qi = lax.broadcasted_iota(jnp.int32, (Sq, Skv_per), 0)
        kj = lax.broadcasted_iota(jnp.int32, (Sq, Skv_per), 1)

        for r in rdmas:
            r.wait_recv()

        for b in range(B):
            ctx_heads = []
            for h in range(Hq):
                qh = q_all[b][:, h * Dh:(h + 1) * Dh].astype(_CDT)
                scores = []
                for s in range(N_DEV):
                    origin = lax.rem(my - s + N_DEV, N_DEV)
                    kg = kj + origin * Skv_per
                    kh = kv_ref[s, 0, b][:, h, :]
                    sc = lax.dot_general(
                        qh, kh, (((1,), (1,)), ((), ())),
                        preferred_element_type=jnp.float32,
                    ) * 0.125
                    mask = (jnp.abs(qi - kg) <= 128) | (kg < 32) | (qi < 32)
                    scores.append(jnp.where(mask, sc, -1e9))
                row_max = functools.reduce(
                    jnp.maximum,
                    [sc.max(axis=-1, keepdims=True) for sc in scores],
                )
                ws = [jnp.exp(sc - row_max) for sc in scores]
                denom = functools.reduce(
                    jnp.add, [w.sum(axis=-1, keepdims=True) for w in ws]
                )
                ctx = functools.reduce(jnp.add, [
                    lax.dot(ws[s].astype(_CDT),
                            kv_ref[s, 1, b][:, h, :],
                            preferred_element_type=jnp.float32)
                    for s in range(N_DEV)
                ])
                ctx_heads.append(ctx / denom)
            ctx_b = jnp.concatenate(ctx_heads, axis=1)
            out_ref[b] = lax.dot(ctx_b.astype(_CDT), wo,
                                 preferred_element_type=jnp.float32)

        for r in rdmas:
            r.wait_send()

    return pl.pallas_call(
        body,
        out_shape=jax.ShapeDtypeStruct((B, Sq, D_model), jnp.float32),
        in_specs=[pl.BlockSpec(memory_space=pltpu.VMEM)] * 5,
        out_specs=pl.BlockSpec(memory_space=pltpu.VMEM),
        scratch_shapes=[
            pltpu.VMEM((N_DEV, 2, B, Skv_per, Hq, Dh), _CDT),
            pltpu.SemaphoreType.DMA((3,)),
            pltpu.SemaphoreType.DMA((3,)),
        ],
        compiler_params=pltpu.CompilerParams(collective_id=0),
    )(x, Wq, K_ext, V_ext, Wo)


# device time: 21720 ns/iter; 1.6477x vs baseline; 1.6477x over previous
import functools

import jax
import jax.numpy as jnp
from jax import lax
from jax.experimental import pallas as pl
from jax.experimental.pallas import tpu as pltpu

N_DEV = 4
B, Sq, Skv_per, Hq, Dh = 2, 128, 128, 4, 64
D_model = 512


def kernel(x, Wq, K_ext, V_ext, Wo):
    def body(x_ref, wq_ref, k_ref, v_ref, wo_ref, out_ref,
             kv_ref, send_sems, recv_sems):
        my = lax.axis_index("i")

        barrier = pltpu.get_barrier_semaphore()
        for off in (1, 2, 3):
            peer = lax.rem(my + off, N_DEV)
            pl.semaphore_signal(
                barrier, inc=1,
                device_id=(peer,), device_id_type=pl.DeviceIdType.MESH,
            )
        pl.semaphore_wait(barrier, N_DEV - 1)

        kv_ref[0, 0] = jnp.transpose(
            k_ref[...], (0, 2, 1, 3)).astype(jnp.bfloat16)
        kv_ref[0, 1] = jnp.transpose(
            v_ref[...], (0, 2, 1, 3)).astype(jnp.bfloat16)

        rdmas = []
        for off in (1, 2, 3):
            peer = lax.rem(my + off, N_DEV)
            rdma = pltpu.make_async_remote_copy(
                src_ref=kv_ref.at[0],
                dst_ref=kv_ref.at[off],
                send_sem=send_sems.at[off - 1],
                recv_sem=recv_sems.at[off - 1],
                device_id=(peer,),
                device_id_type=pl.DeviceIdType.MESH,
            )
            rdma.start()
            rdmas.append(rdma)

        wq = wq_ref[...].astype(jnp.bfloat16)
        q_bh = []
        for b in range(B):
            qb = lax.dot(x_ref[b].astype(jnp.bfloat16), wq,
                         preferred_element_type=jnp.float32)
            q_bh.append(jnp.transpose(
                qb.reshape(Sq, Hq, Dh), (1, 0, 2)).astype(jnp.bfloat16))

        qi = lax.broadcasted_iota(jnp.int32, (Sq, Skv_per), 0)
        kj = lax.broadcasted_iota(jnp.int32, (Sq, Skv_per), 1)
        maskadd = []
        for s in range(N_DEV):
            origin = lax.rem(my - s + N_DEV, N_DEV)
            kg = kj + origin * Skv_per
            keep = (jnp.abs(qi - kg) <= 128) | (kg < 32) | (qi < 32)
            maskadd.append(jnp.where(keep, 0.0, -1e9).astype(jnp.float32))

        scores = [[[None] * N_DEV for _ in range(Hq)] for _ in range(B)]
        for s in range(N_DEV):
            if s > 0:
                rdmas[s - 1].wait_recv()
            for b in range(B):
                for h in range(Hq):
                    kh = kv_ref[s, 0, b, h]
                    scores[b][h][s] = lax.dot_general(
                        q_bh[b][h], kh, (((1,), (1,)), ((), ())),
                        preferred_element_type=jnp.float32,
                    ) * 0.125 + maskadd[s]

        v32 = [[kv_ref[s, 1, b].astype(jnp.float32) for b in range(B)]
               for s in range(N_DEV)]

        wo = wo_ref[...]
        for b in range(B):
            outs = []
            for h in range(Hq):
                sc = scores[b][h]
                row_max = functools.reduce(
                    jnp.maximum,
                    [s_.max(axis=-1, keepdims=True) for s_ in sc],
                )
                ws = [jnp.exp(s_ - row_max) for s_ in sc]
                denom = functools.reduce(
                    jnp.add, [w.sum(axis=-1, keepdims=True) for w in ws]
                )
                ctx = functools.reduce(jnp.add, [
                    lax.dot(ws[s], v32[s][b][h],
                            preferred_element_type=jnp.float32)
                    for s in range(N_DEV)
                ]) / denom
                outs.append(lax.dot(ctx, wo[h * Dh:(h + 1) * Dh],
                                    preferred_element_type=jnp.float32))
            out_ref[b] = functools.reduce(jnp.add, outs)

        for r in rdmas:
            r.wait_send()

    return pl.pallas_call(
        body,
        out_shape=jax.ShapeDtypeStruct((B, Sq, D_model), jnp.float32),
        in_specs=[pl.BlockSpec(memory_space=pltpu.VMEM)] * 5,
        out_specs=pl.BlockSpec(memory_space=pltpu.VMEM),
        scratch_shapes=[
            pltpu.VMEM((N_DEV, 2, B, Hq, Skv_per, Dh), jnp.bfloat16),
            pltpu.SemaphoreType.DMA((3,)),
            pltpu.SemaphoreType.DMA((3,)),
        ],
        compiler_params=pltpu.CompilerParams(collective_id=0),
    )(x, Wq, K_ext, V_ext, Wo)


# device time: 19158 ns/iter; 1.8681x vs baseline; 1.1337x over previous
import functools

import jax
import jax.numpy as jnp
from jax import lax
from jax.experimental import pallas as pl
from jax.experimental.pallas import tpu as pltpu

N_DEV = 4
B, Sq, Skv_per, Hq, Dh = 2, 128, 128, 4, 64
D_model = 512


def kernel(x, Wq, K_ext, V_ext, Wo):
    def body(x_ref, wq_ref, k_ref, v_ref, wo_ref, out_ref,
             kv_ref, ksend, krecv, vsend, vrecv):
        my = lax.axis_index("i")

        barrier = pltpu.get_barrier_semaphore()
        for off in (1, 2, 3):
            peer = lax.rem(my + off, N_DEV)
            pl.semaphore_signal(
                barrier, inc=1,
                device_id=(peer,), device_id_type=pl.DeviceIdType.MESH,
            )
        pl.semaphore_wait(barrier, N_DEV - 1)

        def push(kv, send_sems, recv_sems):
            out = []
            for off in (1, 2, 3):
                peer = lax.rem(my + off, N_DEV)
                rdma = pltpu.make_async_remote_copy(
                    src_ref=kv_ref.at[0, kv],
                    dst_ref=kv_ref.at[off, kv],
                    send_sem=send_sems.at[off - 1],
                    recv_sem=recv_sems.at[off - 1],
                    device_id=(peer,),
                    device_id_type=pl.DeviceIdType.MESH,
                )
                rdma.start()
                out.append(rdma)
            return out

        kv_ref[0, 0] = jnp.transpose(
            k_ref[...], (0, 2, 1, 3)).astype(jnp.bfloat16)
        k_rdmas = push(0, ksend, krecv)
        kv_ref[0, 1] = jnp.transpose(
            v_ref[...], (0, 2, 1, 3)).astype(jnp.bfloat16)
        v_rdmas = push(1, vsend, vrecv)

        wq = wq_ref[...].astype(jnp.bfloat16)
        q_bh = []
        for b in range(B):
            qb = lax.dot(x_ref[b].astype(jnp.bfloat16), wq,
                         preferred_element_type=jnp.float32)
            q_bh.append(jnp.transpose(
                qb.reshape(Sq, Hq, Dh), (1, 0, 2)).astype(jnp.bfloat16))

        qi = lax.broadcasted_iota(jnp.int32, (Sq, Skv_per), 0)
        kj = lax.broadcasted_iota(jnp.int32, (Sq, Skv_per), 1)
        maskadd = []
        for s in range(N_DEV):
            origin = lax.rem(my - s + N_DEV, N_DEV)
            kg = kj + origin * Skv_per
            keep = (jnp.abs(qi - kg) <= 128) | (kg < 32) | (qi < 32)
            maskadd.append(jnp.where(keep, 0.0, -1e9).astype(jnp.float32))

        scores = [[[None] * N_DEV for _ in range(Hq)] for _ in range(B)]
        for s in range(N_DEV):
            if s > 0:
                k_rdmas[s - 1].wait_recv()
            for b in range(B):
                for h in range(Hq):
                    kh = kv_ref[s, 0, b, h]
                    scores[b][h][s] = lax.dot_general(
                        q_bh[b][h], kh, (((1,), (1,)), ((), ())),
                        preferred_element_type=jnp.float32,
                    ) * 0.125 + maskadd[s]

        ws = [[None] * Hq for _ in range(B)]
        inv_denom = [[None] * Hq for _ in range(B)]
        for b in range(B):
            for h in range(Hq):
                sc = scores[b][h]
                row_max = functools.reduce(
                    jnp.maximum,
                    [s_.max(axis=-1, keepdims=True) for s_ in sc],
                )
                w = [jnp.exp(s_ - row_max) for s_ in sc]
                denom = functools.reduce(
                    jnp.add, [w_.sum(axis=-1, keepdims=True) for w_ in w]
                )
                ws[b][h] = [w_.astype(jnp.bfloat16) for w_ in w]
                inv_denom[b][h] = 1.0 / denom

        ctx = [[None] * Hq for _ in range(B)]
        for s in range(N_DEV):
            if s > 0:
                v_rdmas[s - 1].wait_recv()
            for b in range(B):
                for h in range(Hq):
                    part = lax.dot(ws[b][h][s], kv_ref[s, 1, b, h],
                                   preferred_element_type=jnp.float32)
                    ctx[b][h] = part if s == 0 else ctx[b][h] + part

        wo = wo_ref[...]
        for b in range(B):
            out_ref[b] = functools.reduce(jnp.add, [
                lax.dot(ctx[b][h] * inv_denom[b][h],
                        wo[h * Dh:(h + 1) * Dh],
                        preferred_element_type=jnp.float32)
                for h in range(Hq)
            ])

        for r in k_rdmas + v_rdmas:
            r.wait_send()

    return pl.pallas_call(
        body,
        out_shape=jax.ShapeDtypeStruct((B, Sq, D_model), jnp.float32),
        in_specs=[pl.BlockSpec(memory_space=pltpu.VMEM)] * 5,
        out_specs=pl.BlockSpec(memory_space=pltpu.VMEM),
        scratch_shapes=[
            pltpu.VMEM((N_DEV, 2, B, Hq, Skv_per, Dh), jnp.bfloat16),
            pltpu.SemaphoreType.DMA((3,)),
            pltpu.SemaphoreType.DMA((3,)),
            pltpu.SemaphoreType.DMA((3,)),
            pltpu.SemaphoreType.DMA((3,)),
        ],
        compiler_params=pltpu.CompilerParams(collective_id=0),
    )(x, Wq, K_ext, V_ext, Wo)


# device time: 16421 ns/iter; 2.1795x vs baseline; 1.1667x over previous
import functools

import jax
import jax.numpy as jnp
from jax import lax
from jax.experimental import pallas as pl
from jax.experimental.pallas import tpu as pltpu

N_DEV = 4
B, Sq, Skv_per, Hq, Dh = 2, 128, 128, 4, 64
D_model = 512


def kernel(x, Wq, K_ext, V_ext, Wo):
    def body(x_ref, wq_ref, k_ref, v_ref, wo_ref, out_ref,
             msg_ref, den_ref, msend, mrecv, dsend, drecv):
        my = lax.axis_index("i")

        barrier = pltpu.get_barrier_semaphore()
        for off in (1, 2, 3):
            peer = lax.rem(my + off, N_DEV)
            pl.semaphore_signal(
                barrier, inc=1,
                device_id=(peer,), device_id_type=pl.DeviceIdType.MESH,
            )
        pl.semaphore_wait(barrier, N_DEV - 1)

        kT = jnp.transpose(k_ref[...], (0, 2, 1, 3)).astype(jnp.bfloat16)
        vT = jnp.transpose(v_ref[...], (0, 2, 1, 3)).astype(jnp.bfloat16)

        wq = wq_ref[...].astype(jnp.bfloat16)
        q_bh = []
        for b in range(B):
            qb = lax.dot(x_ref[b].astype(jnp.bfloat16), wq,
                         preferred_element_type=jnp.float32)
            q_bh.append(jnp.transpose(
                qb.reshape(Sq, Hq, Dh), (1, 0, 2)).astype(jnp.bfloat16))

        qi = lax.broadcasted_iota(jnp.int32, (Sq, Skv_per), 0)
        kj = lax.broadcasted_iota(jnp.int32, (Sq, Skv_per), 1)
        kg = kj + my * Skv_per
        keep = (jnp.abs(qi - kg) <= 128) | (kg < 32) | (qi < 32)
        maskadd = jnp.where(keep, 0.0, -1e9).astype(jnp.float32)

        ctx_acc = [[None] * Hq for _ in range(B)]
        den_acc = [[None] * Hq for _ in range(B)]
        for b in range(B):
            for h in range(Hq):
                sc = lax.dot_general(
                    q_bh[b][h], kT[b, h], (((1,), (1,)), ((), ())),
                    preferred_element_type=jnp.float32,
                ) * 0.125 + maskadd
                w = jnp.exp(sc)
                den = w.sum(axis=-1, keepdims=True)
                ctx = lax.dot(w.astype(jnp.bfloat16), vT[b, h],
                              preferred_element_type=jnp.float32)
                msg_ref[0, b, h] = ctx.astype(jnp.bfloat16)
                den_ref[0, b, h] = den.reshape(Sq)
                ctx_acc[b][h] = ctx
                den_acc[b][h] = den

        ctx_rdmas, den_rdmas = [], []
        for off in (1, 2, 3):
            peer = lax.rem(my + off, N_DEV)
            for src, dst, ss, rs, lst in (
                (msg_ref.at[0], msg_ref.at[off], msend, mrecv, ctx_rdmas),
                (den_ref.at[0], den_ref.at[off], dsend, drecv, den_rdmas),
            ):
                rdma = pltpu.make_async_remote_copy(
                    src_ref=src, dst_ref=dst,
                    send_sem=ss.at[off - 1], recv_sem=rs.at[off - 1],
                    device_id=(peer,),
                    device_id_type=pl.DeviceIdType.MESH,
                )
                rdma.start()
                lst.append(rdma)

        wo = wo_ref[...]

        for s in range(1, N_DEV):
            ctx_rdmas[s - 1].wait_recv()
            den_rdmas[s - 1].wait_recv()
            for b in range(B):
                for h in range(Hq):
                    ctx_acc[b][h] = ctx_acc[b][h] + \
                        msg_ref[s, b, h].astype(jnp.float32)
                    den_acc[b][h] = den_acc[b][h] + \
                        den_ref[s, b, h].reshape(Sq, 1)

        for b in range(B):
            out_ref[b] = functools.reduce(jnp.add, [
                lax.dot(ctx_acc[b][h] / den_acc[b][h],
                        wo[h * Dh:(h + 1) * Dh],
                        preferred_element_type=jnp.float32)
                for h in range(Hq)
            ])

        for r in ctx_rdmas + den_rdmas:
            r.wait_send()

    return pl.pallas_call(
        body,
        out_shape=jax.ShapeDtypeStruct((B, Sq, D_model), jnp.float32),
        in_specs=[pl.BlockSpec(memory_space=pltpu.VMEM)] * 5,
        out_specs=pl.BlockSpec(memory_space=pltpu.VMEM),
        scratch_shapes=[
            pltpu.VMEM((N_DEV, B, Hq, Sq, Dh), jnp.bfloat16),
            pltpu.VMEM((N_DEV, B, Hq, Sq), jnp.float32),
            pltpu.SemaphoreType.DMA((3,)),
            pltpu.SemaphoreType.DMA((3,)),
            pltpu.SemaphoreType.DMA((3,)),
            pltpu.SemaphoreType.DMA((3,)),
        ],
        compiler_params=pltpu.CompilerParams(collective_id=0),
    )(x, Wq, K_ext, V_ext, Wo)


# device time: 15500 ns/iter; 2.3090x vs baseline; 1.0594x over previous
import jax
import jax.numpy as jnp
from jax import lax
from jax.experimental import pallas as pl
from jax.experimental.pallas import tpu as pltpu

N_DEV = 4
B, Sq, Skv_per, Hq, Dh = 2, 128, 128, 4, 64
D_model = 512


def kernel(x, Wq, K_ext, V_ext, Wo):
    def body(x_ref, wq_ref, k_ref, v_ref, wo_ref, out_ref,
             msg_ref, den_ref, msend, mrecv, dsend, drecv):
        my = lax.axis_index("i")

        barrier = pltpu.get_barrier_semaphore()
        for off in (1, 2, 3):
            peer = lax.rem(my + off, N_DEV)
            pl.semaphore_signal(
                barrier, inc=1,
                device_id=(peer,), device_id_type=pl.DeviceIdType.MESH,
            )
        pl.semaphore_wait(barrier, N_DEV - 1)

        kT = jnp.transpose(k_ref[...], (0, 2, 1, 3)).astype(jnp.bfloat16)
        vT = jnp.transpose(v_ref[...], (0, 2, 1, 3)).astype(jnp.bfloat16)

        wq = wq_ref[...].astype(jnp.bfloat16)
        qs = []
        for b in range(B):
            qb = lax.dot(x_ref[b].astype(jnp.bfloat16), wq,
                         preferred_element_type=jnp.float32)
            qs.append(jnp.transpose(
                qb.reshape(Sq, Hq, Dh), (1, 0, 2)).astype(jnp.bfloat16))

        qi = lax.broadcasted_iota(jnp.int32, (Sq, Skv_per), 0)
        kj = lax.broadcasted_iota(jnp.int32, (Sq, Skv_per), 1)
        kg = kj + my * Skv_per
        keep = (jnp.abs(qi - kg) <= 128) | (kg < 32) | (qi < 32)
        maskadd = jnp.where(keep, 0.0, -1e9).astype(jnp.float32)

        ctx_acc, den_acc = [None] * B, [None] * B
        ctx_rdmas = [[None] * 3 for _ in range(B)]
        den_rdmas = [[None] * 3 for _ in range(B)]
        for b in range(B):
            sc = lax.dot_general(
                qs[b], kT[b], (((2,), (2,)), ((0,), (0,))),
                preferred_element_type=jnp.float32,
            ) * 0.125 + maskadd[None]
            w = jnp.exp(sc)
            den = w.sum(axis=-1)
            ctx = lax.dot_general(
                w.astype(jnp.bfloat16), vT[b], (((2,), (1,)), ((0,), (0,))),
                preferred_element_type=jnp.float32,
            )
            msg_ref[0, b] = ctx.astype(jnp.bfloat16)
            den_ref[0, b] = den
            ctx_acc[b], den_acc[b] = ctx, den
            for off in (1, 2, 3):
                peer = lax.rem(my + off, N_DEV)
                for src, dst, ss, rs, lst in (
                    (msg_ref, msg_ref, msend, mrecv, ctx_rdmas),
                    (den_ref, den_ref, dsend, drecv, den_rdmas),
                ):
                    rdma = pltpu.make_async_remote_copy(
                        src_ref=src.at[0, b], dst_ref=dst.at[off, b],
                        send_sem=ss.at[b, off - 1],
                        recv_sem=rs.at[b, off - 1],
                        device_id=(peer,),
                        device_id_type=pl.DeviceIdType.MESH,
                    )
                    rdma.start()
                    lst[b][off - 1] = rdma

        wo = wo_ref[...]

        for b in range(B):
            for s in range(1, N_DEV):
                ctx_rdmas[b][s - 1].wait_recv()
                den_rdmas[b][s - 1].wait_recv()
                ctx_acc[b] = ctx_acc[b] + msg_ref[s, b].astype(jnp.float32)
                den_acc[b] = den_acc[b] + den_ref[s, b]
            ctxn = ctx_acc[b] / den_acc[b][:, :, None]
            acc = None
            for h in range(Hq):
                part = lax.dot(ctxn[h], wo[h * Dh:(h + 1) * Dh],
                               preferred_element_type=jnp.float32)
                acc = part if acc is None else acc + part
            out_ref[b] = acc

        for b in range(B):
            for r in ctx_rdmas[b] + den_rdmas[b]:
                r.wait_send()

    return pl.pallas_call(
        body,
        out_shape=jax.ShapeDtypeStruct((B, Sq, D_model), jnp.float32),
        in_specs=[pl.BlockSpec(memory_space=pltpu.VMEM)] * 5,
        out_specs=pl.BlockSpec(memory_space=pltpu.VMEM),
        scratch_shapes=[
            pltpu.VMEM((N_DEV, B, Hq, Sq, Dh), jnp.bfloat16),
            pltpu.VMEM((N_DEV, B, Hq, Sq), jnp.float32),
            pltpu.SemaphoreType.DMA((B, 3)),
            pltpu.SemaphoreType.DMA((B, 3)),
            pltpu.SemaphoreType.DMA((B, 3)),
            pltpu.SemaphoreType.DMA((B, 3)),
        ],
        compiler_params=pltpu.CompilerParams(collective_id=0),
    )(x, Wq, K_ext, V_ext, Wo)


# device time: 12133 ns/iter; 2.9497x vs baseline; 1.2775x over previous
import jax
import jax.numpy as jnp
from jax import lax
from jax.experimental import pallas as pl
from jax.experimental.pallas import tpu as pltpu

N_DEV = 4
B, Sq, Skv_per, Hq, Dh = 2, 128, 128, 4, 64
D_model = 512
_COMM = True


def kernel(x, Wq, K_ext, V_ext, Wo):
    def body(x_ref, wq_ref, k_ref, v_ref, wo_ref, out_ref,
             msg_ref, den_ref, msend, mrecv, dsend, drecv):
        my = lax.axis_index("i")

        if _COMM:
            barrier = pltpu.get_barrier_semaphore()
            for off in (1, 2, 3):
                peer = lax.rem(my + off, N_DEV)
                pl.semaphore_signal(
                    barrier, inc=1,
                    device_id=(peer,), device_id_type=pl.DeviceIdType.MESH,
                )

        wq = wq_ref[...].astype(jnp.bfloat16)

        qi = lax.broadcasted_iota(jnp.int32, (Sq, Skv_per), 0)
        kj = lax.broadcasted_iota(jnp.int32, (Sq, Skv_per), 1)
        kg = kj + my * Skv_per
        keep = (jnp.abs(qi - kg) <= 128) | (kg < 32) | (qi < 32)
        maskadd = jnp.where(keep, 0.0, -1e9).astype(jnp.float32)

        ctx_acc, den_acc = [None] * B, [None] * B
        ctx_rdmas = [[None] * 3 for _ in range(B)]
        den_rdmas = [[None] * 3 for _ in range(B)]
        for b in range(B):
            kTb = jnp.transpose(k_ref[b], (1, 0, 2)).astype(jnp.bfloat16)
            vTb = jnp.transpose(v_ref[b], (1, 0, 2)).astype(jnp.bfloat16)
            qTb = (lax.dot_general(
                wq, x_ref[b].astype(jnp.bfloat16),
                (((0,), (1,)), ((), ())),
                preferred_element_type=jnp.float32,
            ) * 0.125).astype(jnp.bfloat16).reshape(Hq, Dh, Sq)
            sc = lax.dot_general(
                qTb, kTb, (((1,), (2,)), ((0,), (0,))),
                preferred_element_type=jnp.float32,
            ) + maskadd[None]
            w = jnp.exp(sc)
            den = w.sum(axis=-1)
            ctx = lax.dot_general(
                vTb, w.astype(jnp.bfloat16), (((1,), (2,)), ((0,), (0,))),
                preferred_element_type=jnp.float32,
            )
            msg_ref[0, b] = ctx.astype(jnp.bfloat16)
            den_ref[0, b] = den
            ctx_acc[b], den_acc[b] = ctx, den
            if _COMM and b == 0:
                pl.semaphore_wait(barrier, N_DEV - 1)
            for off in (1, 2, 3) if _COMM else ():
                peer = lax.rem(my + off, N_DEV)
                for src, dst, ss, rs, lst in (
                    (msg_ref, msg_ref, msend, mrecv, ctx_rdmas),
                    (den_ref, den_ref, dsend, drecv, den_rdmas),
                ):
                    rdma = pltpu.make_async_remote_copy(
                        src_ref=src.at[0, b], dst_ref=dst.at[off, b],
                        send_sem=ss.at[b, off - 1],
                        recv_sem=rs.at[b, off - 1],
                        device_id=(peer,),
                        device_id_type=pl.DeviceIdType.MESH,
                    )
                    rdma.start()
                    lst[b][off - 1] = rdma

        wo = wo_ref[...]

        for b in range(B):
            for s in range(1, N_DEV):
                if _COMM:
                    ctx_rdmas[b][s - 1].wait_recv()
                    den_rdmas[b][s - 1].wait_recv()
                ctx_acc[b] = ctx_acc[b] + msg_ref[s, b].astype(jnp.float32)
                den_acc[b] = den_acc[b] + den_ref[s, b]
            ctxn = ctx_acc[b] / den_acc[b][:, None, :]
            acc = None
            for h in range(Hq):
                part = lax.dot_general(
                    ctxn[h], wo[h * Dh:(h + 1) * Dh],
                    (((0,), (0,)), ((), ())),
                    preferred_element_type=jnp.float32)
                acc = part if acc is None else acc + part
            out_ref[b] = acc

        for b in range(B) if _COMM else ():
            for r in ctx_rdmas[b] + den_rdmas[b]:
                r.wait_send()

    return pl.pallas_call(
        body,
        out_shape=jax.ShapeDtypeStruct((B, Sq, D_model), jnp.float32),
        in_specs=[pl.BlockSpec(memory_space=pltpu.VMEM)] * 5,
        out_specs=pl.BlockSpec(memory_space=pltpu.VMEM),
        scratch_shapes=[
            pltpu.VMEM((N_DEV, B, Hq, Dh, Sq), jnp.bfloat16),
            pltpu.VMEM((N_DEV, B, Hq, Sq), jnp.float32),
            pltpu.SemaphoreType.DMA((B, 3)),
            pltpu.SemaphoreType.DMA((B, 3)),
            pltpu.SemaphoreType.DMA((B, 3)),
            pltpu.SemaphoreType.DMA((B, 3)),
        ],
        compiler_params=(
            pltpu.CompilerParams(collective_id=0) if _COMM
            else pltpu.CompilerParams()
        ),
    )(x, Wq, K_ext, V_ext, Wo)
